# Initial kernel scaffold; baseline (speedup 1.0000x reference)
import jax
import jax.numpy as jnp
from jax.experimental import pallas as pl


def kernel(hidden_states, gate_w, w_gate, w_up, w_down):
    raise NotImplementedError("write your pallas kernel here")


# speedup vs baseline: 1.0340x; 1.0340x over previous
import functools

import jax
import jax.numpy as jnp
from jax import lax
from jax.experimental import pallas as pl
from jax.experimental.pallas import tpu as pltpu
from jax.experimental.pallas import tpu_sc as plsc

E = 8
K = 2
D = 1024
F = 512
T = 4096
A = T * K
BLK = 256
PAD = A + E * BLK
NBLK = PAD // BLK
CHK = 512
NCH = T // CHK

NC, NS = 2, 16
NW = NC * NS
APW = A // NW
RPW = PAD // NW
TPW = T // NW
GCH = 64
CCH = 32

@functools.lru_cache(maxsize=None)
def _sc_mesh():
    return plsc.VectorSubcoreMesh(
        core_axis_name="c", subcore_axis_name="s",
        num_cores=NC, num_subcores=NS)


def _worker_id():
    return lax.axis_index("s") * NC + lax.axis_index("c")


def _router_body(x_ref, gw_ref, pos0_ref, pos1_ref, tw0_ref, tw1_ref, be_ref):
    x = x_ref[...]
    gw = gw_ref[...]
    logits = lax.dot_general(x, gw, (((1,), (1,)), ((), ())),
                             preferred_element_type=jnp.float32)
    scores = jax.nn.sigmoid(logits)
    ie = lax.broadcasted_iota(jnp.int32, (T, E), 1)
    m1 = jnp.max(scores, axis=1, keepdims=True)
    e0 = jnp.min(jnp.where(scores >= m1, ie, E), axis=1, keepdims=True)
    oh0 = ie == e0
    s2 = jnp.where(oh0, -1.0, scores)
    m2 = jnp.max(s2, axis=1, keepdims=True)
    e1 = jnp.min(jnp.where(s2 >= m2, ie, E), axis=1, keepdims=True)
    oh1 = ie == e1
    den = m1 + m2 + 1e-20
    tw0_ref[...] = m1 / den
    tw1_ref[...] = m2 / den

    ind0 = oh0.astype(jnp.float32)
    ind1 = oh1.astype(jnp.float32)
    ri = lax.broadcasted_iota(jnp.int32, (CHK, CHK), 0)
    ci = lax.broadcasted_iota(jnp.int32, (CHK, CHK), 1)
    tstrict = (ci < ri).astype(jnp.float32)

    def chunk_ranks(ind):
        pref = jnp.zeros((1, E), jnp.float32)
        sls, rks = [], []
        for c in range(NCH):
            sl = lax.slice_in_dim(ind, c * CHK, (c + 1) * CHK, axis=0)
            loc = lax.dot_general(tstrict, sl, (((1,), (0,)), ((), ())),
                                  preferred_element_type=jnp.float32)
            rks.append(jnp.sum(sl * (loc + pref), axis=1, keepdims=True))
            sls.append(sl)
            pref = pref + jnp.sum(sl, axis=0, keepdims=True)
        return sls, rks, pref

    sl0, rk0, cnt0 = chunk_ranks(ind0)
    sl1, rk1, cnt1 = chunk_ranks(ind1)
    counts = cnt0 + cnt1
    seg = jnp.floor((counts + (BLK - 1)) * (1.0 / BLK)) * BLK
    ea = lax.broadcasted_iota(jnp.int32, (E, E), 0)
    eb = lax.broadcasted_iota(jnp.int32, (E, E), 1)
    upper = (ea < eb).astype(jnp.float32)
    offs = lax.dot_general(seg, upper, (((1,), (0,)), ((), ())))
    base1 = offs + cnt0
    for c in range(NCH):
        o0 = jnp.sum(sl0[c] * offs, axis=1, keepdims=True)
        pos0_ref[pl.ds(c * CHK, CHK), :] = (o0 + rk0[c]).astype(jnp.int32)
        o1 = jnp.sum(sl1[c] * base1, axis=1, keepdims=True)
        pos1_ref[pl.ds(c * CHK, CHK), :] = (o1 + rk1[c]).astype(jnp.int32)
    ends = offs + seg
    bi = (lax.broadcasted_iota(jnp.int32, (NBLK, 1), 0) * BLK
          ).astype(jnp.float32)
    be = jnp.sum((bi >= ends).astype(jnp.float32), axis=1, keepdims=True)
    be_ref[...] = jnp.minimum(be, E - 1.0).astype(jnp.int32)


def _router(x, gate_w):
    return pl.pallas_call(
        _router_body,
        out_shape=[
            jax.ShapeDtypeStruct((T, 1), jnp.int32),
            jax.ShapeDtypeStruct((T, 1), jnp.int32),
            jax.ShapeDtypeStruct((T, 1), jnp.float32),
            jax.ShapeDtypeStruct((T, 1), jnp.float32),
            jax.ShapeDtypeStruct((NBLK, 1), jnp.int32),
        ],
    )(x, gate_w)


def _sc_scatter_body(pos2_ref, tw2_ref, ts_ref, ss_ref, pvw, tww, vals):
    wid = _worker_id()
    row0 = wid * (APW // 128)
    pltpu.sync_copy(pos2_ref.at[pl.ds(row0, APW // 128)], pvw)
    pltpu.sync_copy(tw2_ref.at[pl.ds(row0, APW // 128)], tww)
    lane = lax.broadcasted_iota(jnp.int32, (16,), 0)
    for j2 in range(APW // 128):
        for i in range(8):
            jbase = wid * APW + j2 * 128 + i * 16
            vals[j2, pl.ds(i * 16, 16)] = (jbase + lane) & (T - 1)
    for j2 in range(APW // 128):
        pltpu.sync_copy(vals.at[j2], ts_ref.at[pvw.at[j2]])
        pltpu.sync_copy(tww.at[j2], ss_ref.at[pvw.at[j2]])


@functools.lru_cache(maxsize=None)
def _sc_scatter():
    return pl.kernel(
        _sc_scatter_body,
        out_type=[
            jax.ShapeDtypeStruct((PAD,), jnp.int32),
            jax.ShapeDtypeStruct((PAD,), jnp.float32),
        ],
        mesh=_sc_mesh(),
        scratch_types=[
            pltpu.VMEM((APW // 128, 128), jnp.int32),
            pltpu.VMEM((APW // 128, 128), jnp.float32),
            pltpu.VMEM((APW // 128, 128), jnp.int32),
        ],
    )


def _sc_gather_body(ts_ref, x_ref, xd_ref, tsw, rowbuf, sem):
    wid = _worker_id()
    base = pl.multiple_of(wid * RPW, 8)
    pltpu.sync_copy(ts_ref.at[pl.ds(base, RPW)], tsw)

    def clamp(i, carry):
        off = pl.multiple_of(i * 16, 16)
        tsw[pl.ds(off, 16)] = jnp.clip(tsw[pl.ds(off, 16)], 0, T - 1)
        return carry

    lax.fori_loop(0, RPW // 16, clamp, 0)
    for c in range(RPW // GCH):
        idx = tsw.at[pl.ds(c * GCH, GCH)]
        pltpu.async_copy(x_ref.at[idx], rowbuf, sem).wait()
        pltpu.sync_copy(rowbuf, xd_ref.at[pl.ds(base + c * GCH, GCH)])


@functools.lru_cache(maxsize=None)
def _sc_gather():
    return pl.kernel(
        _sc_gather_body,
        out_type=jax.ShapeDtypeStruct((PAD, D), jnp.float32),
        mesh=_sc_mesh(),
        scratch_types=[
            pltpu.VMEM((RPW,), jnp.int32),
            pltpu.VMEM((GCH, D), jnp.float32),
            pltpu.SemaphoreType.DMA,
        ],
    )


def _ffn_body(be_ref, xd_ref, wg_ref, wu_ref, wd_ref, ss_ref, y_ref):
    xb = xd_ref[...]
    g = lax.dot_general(xb, wg_ref[0], (((1,), (1,)), ((), ())),
                        preferred_element_type=jnp.float32)
    u = lax.dot_general(xb, wu_ref[0], (((1,), (1,)), ((), ())),
                        preferred_element_type=jnp.float32)
    h = g * jax.nn.sigmoid(g) * u
    y = lax.dot_general(h, wd_ref[0], (((1,), (1,)), ((), ())),
                        preferred_element_type=jnp.float32)
    y_ref[...] = y * ss_ref[...]


def _ffn(be, xd, w_gate, w_up, w_down, ss):
    grid_spec = pltpu.PrefetchScalarGridSpec(
        num_scalar_prefetch=1,
        grid=(NBLK,),
        in_specs=[
            pl.BlockSpec((BLK, D), lambda i, be: (i, 0)),
            pl.BlockSpec((1, F, D), lambda i, be: (be[i], 0, 0)),
            pl.BlockSpec((1, F, D), lambda i, be: (be[i], 0, 0)),
            pl.BlockSpec((1, D, F), lambda i, be: (be[i], 0, 0)),
            pl.BlockSpec((BLK, 1), lambda i, be: (i, 0)),
        ],
        out_specs=pl.BlockSpec((BLK, D), lambda i, be: (i, 0)),
    )
    return pl.pallas_call(
        _ffn_body,
        grid_spec=grid_spec,
        out_shape=jax.ShapeDtypeStruct((PAD, D), jnp.float32),
        compiler_params=pltpu.CompilerParams(
            dimension_semantics=("arbitrary",),
        ),
    )(be, xd, w_gate, w_up, w_down, ss)


def _sc_combine_body(p02_ref, p12_ref, y_ref, out_ref, p0w, p1w, ya, yb,
                     sema, semb):
    wid = _worker_id()
    pltpu.sync_copy(p02_ref.at[wid], p0w)
    pltpu.sync_copy(p12_ref.at[wid], p1w)
    tok0 = pl.multiple_of(wid * TPW, 8)
    for c in range(TPW // CCH):
        ia = p0w.at[pl.ds(c * CCH, CCH)]
        ib = p1w.at[pl.ds(c * CCH, CCH)]
        ca = pltpu.async_copy(y_ref.at[ia], ya, sema)
        cb = pltpu.async_copy(y_ref.at[ib], yb, semb)
        ca.wait()
        cb.wait()

        def add(j, carry):
            off = pl.multiple_of(j * 16, 16)
            for r in range(CCH):
                ya[r, pl.ds(off, 16)] = (ya[r, pl.ds(off, 16)]
                                         + yb[r, pl.ds(off, 16)])
            return carry

        lax.fori_loop(0, D // 16, add, 0)
        pltpu.sync_copy(ya, out_ref.at[pl.ds(tok0 + c * CCH, CCH)])


@functools.lru_cache(maxsize=None)
def _sc_combine():
    return pl.kernel(
        _sc_combine_body,
        out_type=jax.ShapeDtypeStruct((T, D), jnp.float32),
        mesh=_sc_mesh(),
        scratch_types=[
            pltpu.VMEM((TPW,), jnp.int32),
            pltpu.VMEM((TPW,), jnp.int32),
            pltpu.VMEM((CCH, D), jnp.float32),
            pltpu.VMEM((CCH, D), jnp.float32),
            pltpu.SemaphoreType.DMA,
            pltpu.SemaphoreType.DMA,
        ],
    )


def kernel(hidden_states, gate_w, w_gate, w_up, w_down):
    B, S, d = hidden_states.shape
    x = hidden_states.reshape(-1, d)
    pos0, pos1, tw0, tw1, be = _router(x, gate_w)
    pos_all = jnp.concatenate([pos0, pos1], axis=0).reshape(A // 128, 128)
    tw_all = jnp.concatenate([tw0, tw1], axis=0).reshape(A // 128, 128)
    ts, ss = _sc_scatter()(pos_all, tw_all)
    xd = _sc_gather()(ts, x)
    y = _ffn(be.reshape(NBLK), xd, w_gate, w_up, w_down, ss.reshape(PAD, 1))
    out = _sc_combine()(pos0.reshape(NW, TPW), pos1.reshape(NW, TPW), y)
    return out.reshape(B, S, d)


# speedup vs baseline: 1.5388x; 1.4882x over previous
import functools

import jax
import jax.numpy as jnp
from jax import lax
from jax.experimental import pallas as pl
from jax.experimental.pallas import tpu as pltpu
from jax.experimental.pallas import tpu_sc as plsc

E = 8
K = 2
D = 1024
F = 512
T = 4096
A = T * K
BLK = 256
PAD = A + E * BLK
NBLK = PAD // BLK
CHK = 512
NCH = T // CHK

NC, NS = 2, 16
NW = NC * NS
RPW = PAD // NW
TPW = T // NW
GCH = 32
CCH = 16


@functools.lru_cache(maxsize=None)
def _sc_mesh():
    return plsc.VectorSubcoreMesh(
        core_axis_name="c", subcore_axis_name="s",
        num_cores=NC, num_subcores=NS)


def _worker_id():
    return lax.axis_index("s") * NC + lax.axis_index("c")


def _router_body(x_ref, gw_ref, pos0_ref, pos1_ref, tw0_ref, tw1_ref, be_ref):
    x = x_ref[...]
    gw = gw_ref[...]
    logits = lax.dot_general(x, gw, (((1,), (1,)), ((), ())),
                             preferred_element_type=jnp.float32)
    scores = jax.nn.sigmoid(logits)
    ie = lax.broadcasted_iota(jnp.int32, (T, E), 1)
    m1 = jnp.max(scores, axis=1, keepdims=True)
    e0 = jnp.min(jnp.where(scores >= m1, ie, E), axis=1, keepdims=True)
    oh0 = ie == e0
    s2 = jnp.where(oh0, -1.0, scores)
    m2 = jnp.max(s2, axis=1, keepdims=True)
    e1 = jnp.min(jnp.where(s2 >= m2, ie, E), axis=1, keepdims=True)
    oh1 = ie == e1
    den = m1 + m2 + 1e-20
    tw0_ref[...] = m1 / den
    tw1_ref[...] = m2 / den

    ind0 = oh0.astype(jnp.float32)
    ind1 = oh1.astype(jnp.float32)
    ri = lax.broadcasted_iota(jnp.int32, (CHK, CHK), 0)
    ci = lax.broadcasted_iota(jnp.int32, (CHK, CHK), 1)
    tstrict = (ci < ri).astype(jnp.float32)

    def chunk_ranks(ind):
        pref = jnp.zeros((1, E), jnp.float32)
        sls, rks = [], []
        for c in range(NCH):
            sl = lax.slice_in_dim(ind, c * CHK, (c + 1) * CHK, axis=0)
            loc = lax.dot_general(tstrict, sl, (((1,), (0,)), ((), ())),
                                  preferred_element_type=jnp.float32)
            rks.append(jnp.sum(sl * (loc + pref), axis=1, keepdims=True))
            sls.append(sl)
            pref = pref + jnp.sum(sl, axis=0, keepdims=True)
        return sls, rks, pref

    sl0, rk0, cnt0 = chunk_ranks(ind0)
    sl1, rk1, cnt1 = chunk_ranks(ind1)
    counts = cnt0 + cnt1
    seg = jnp.floor((counts + (BLK - 1)) * (1.0 / BLK)) * BLK
    ea = lax.broadcasted_iota(jnp.int32, (E, E), 0)
    eb = lax.broadcasted_iota(jnp.int32, (E, E), 1)
    upper = (ea < eb).astype(jnp.float32)
    offs = lax.dot_general(seg, upper, (((1,), (0,)), ((), ())))
    base1 = offs + cnt0
    for c in range(NCH):
        o0 = jnp.sum(sl0[c] * offs, axis=1, keepdims=True)
        pos0_ref[pl.ds(c * CHK, CHK), :] = (o0 + rk0[c]).astype(jnp.int32)
        o1 = jnp.sum(sl1[c] * base1, axis=1, keepdims=True)
        pos1_ref[pl.ds(c * CHK, CHK), :] = (o1 + rk1[c]).astype(jnp.int32)
    ends = offs + seg
    bi = (lax.broadcasted_iota(jnp.int32, (NBLK, 1), 0) * BLK
          ).astype(jnp.float32)
    be = jnp.sum((bi >= ends).astype(jnp.float32), axis=1, keepdims=True)
    be_ref[...] = jnp.minimum(be, E - 1.0).astype(jnp.int32)


def _router(x, gate_w):
    return pl.pallas_call(
        _router_body,
        out_shape=[
            jax.ShapeDtypeStruct((T, 1), jnp.int32),
            jax.ShapeDtypeStruct((T, 1), jnp.int32),
            jax.ShapeDtypeStruct((T, 1), jnp.float32),
            jax.ShapeDtypeStruct((T, 1), jnp.float32),
            jax.ShapeDtypeStruct((NBLK, 1), jnp.int32),
        ],
    )(x, gate_w)


DCH = 32
NCHD = (A // NW) // DCH


def _sc_dispatch_body(pos3_ref, tw3_ref, x_ref, xd_ref, ss_ref,
                      pvw, tww, buf0, buf1, gs0, gs1, ws0, ws1, ssem):
    wid = _worker_id()
    row0 = wid * NCHD
    pltpu.sync_copy(pos3_ref.at[pl.ds(row0, NCHD)], pvw)
    pltpu.sync_copy(tw3_ref.at[pl.ds(row0, NCHD)], tww)
    sdescs = []
    for j in range(NCHD):
        sdescs.append(
            pltpu.async_copy(tww.at[j], ss_ref.at[pvw.at[j]], ssem))
    t0 = (wid * (A // NW)) & (T - 1)

    bufs, gsems, wsems = (buf0, buf1), (gs0, gs1), (ws0, ws1)
    gd = [None, None]
    wd = [None, None]

    def start(c):
        b = c & 1
        gd[b] = pltpu.async_copy(
            x_ref.at[pl.ds(pl.multiple_of(t0 + c * DCH, 8), DCH)],
            bufs[b], gsems[b])

    start(0)
    for c in range(NCHD):
        b = c & 1
        gd[b].wait()
        if c + 1 < NCHD:
            if wd[1 - b] is not None:
                wd[1 - b].wait()
            start(c + 1)
        wd[b] = pltpu.async_copy(bufs[b], xd_ref.at[pvw.at[c]], wsems[b])
    wd[0].wait()
    wd[1].wait()
    for sd in sdescs:
        sd.wait()


@functools.lru_cache(maxsize=None)
def _sc_dispatch():
    return pl.kernel(
        _sc_dispatch_body,
        out_type=[
            jax.ShapeDtypeStruct((PAD, D), jnp.float32),
            jax.ShapeDtypeStruct((PAD,), jnp.float32),
        ],
        mesh=_sc_mesh(),
        scratch_types=[
            pltpu.VMEM((NCHD, DCH), jnp.int32),
            pltpu.VMEM((NCHD, DCH), jnp.float32),
            pltpu.VMEM((DCH, D), jnp.float32),
            pltpu.VMEM((DCH, D), jnp.float32),
            pltpu.SemaphoreType.DMA,
            pltpu.SemaphoreType.DMA,
            pltpu.SemaphoreType.DMA,
            pltpu.SemaphoreType.DMA,
            pltpu.SemaphoreType.DMA,
        ],
    )


def _ffn_body(be_ref, xd_ref, wg_ref, wu_ref, wd_ref, ss_ref, y_ref):
    xb = xd_ref[...].astype(jnp.bfloat16)
    g = lax.dot_general(xb, wg_ref[0], (((1,), (1,)), ((), ())),
                        preferred_element_type=jnp.float32)
    u = lax.dot_general(xb, wu_ref[0], (((1,), (1,)), ((), ())),
                        preferred_element_type=jnp.float32)
    h = (g * jax.nn.sigmoid(g) * u).astype(jnp.bfloat16)
    y = lax.dot_general(h, wd_ref[0], (((1,), (1,)), ((), ())),
                        preferred_element_type=jnp.float32)
    y_ref[...] = y * ss_ref[...]


def _ffn(be, xd, w_gate, w_up, w_down, ss):
    grid_spec = pltpu.PrefetchScalarGridSpec(
        num_scalar_prefetch=1,
        grid=(NBLK,),
        in_specs=[
            pl.BlockSpec((BLK, D), lambda i, be: (i, 0)),
            pl.BlockSpec((1, F, D), lambda i, be: (be[i], 0, 0)),
            pl.BlockSpec((1, F, D), lambda i, be: (be[i], 0, 0)),
            pl.BlockSpec((1, D, F), lambda i, be: (be[i], 0, 0)),
            pl.BlockSpec((BLK, 1), lambda i, be: (i, 0)),
        ],
        out_specs=pl.BlockSpec((BLK, D), lambda i, be: (i, 0)),
    )
    return pl.pallas_call(
        _ffn_body,
        grid_spec=grid_spec,
        out_shape=jax.ShapeDtypeStruct((PAD, D), jnp.float32),
        compiler_params=pltpu.CompilerParams(
            dimension_semantics=("arbitrary",),
        ),
    )(be, xd, w_gate, w_up, w_down, ss)


def _sc_combine_body(p02_ref, p12_ref, y_ref, out_ref, p0w, p1w,
                     ya0, yb0, ya1, yb1, ga0, gb0, ga1, gb1, ws0, ws1):
    wid = _worker_id()
    tok0 = pl.multiple_of(wid * TPW, 8)
    pltpu.sync_copy(p02_ref.at[wid], p0w)
    pltpu.sync_copy(p12_ref.at[wid], p1w)
    yas, ybs = (ya0, ya1), (yb0, yb1)
    gas, gbs, wss = (ga0, ga1), (gb0, gb1), (ws0, ws1)
    gda = [None, None]
    gdb = [None, None]
    wd = [None, None]
    NCC = TPW // CCH

    def start(c):
        b = c & 1
        ia = p0w.at[pl.ds(c * CCH, CCH)]
        ib = p1w.at[pl.ds(c * CCH, CCH)]
        gda[b] = pltpu.async_copy(y_ref.at[ia], yas[b], gas[b])
        gdb[b] = pltpu.async_copy(y_ref.at[ib], ybs[b], gbs[b])

    start(0)
    for c in range(NCC):
        b = c & 1
        gda[b].wait()
        gdb[b].wait()
        if c + 1 < NCC:
            if wd[1 - b] is not None:
                wd[1 - b].wait()
            start(c + 1)
        ya, yb = yas[b], ybs[b]

        def add(j, carry):
            off = pl.multiple_of(j * 16, 16)
            for r in range(CCH):
                ya[r, pl.ds(off, 16)] = (ya[r, pl.ds(off, 16)]
                                         + yb[r, pl.ds(off, 16)])
            return carry

        lax.fori_loop(0, D // 16, add, 0)
        wd[b] = pltpu.async_copy(
            ya, out_ref.at[pl.ds(tok0 + c * CCH, CCH)], wss[b])
    wd[0].wait()
    wd[1].wait()


@functools.lru_cache(maxsize=None)
def _sc_combine():
    return pl.kernel(
        _sc_combine_body,
        out_type=jax.ShapeDtypeStruct((T, D), jnp.float32),
        mesh=_sc_mesh(),
        scratch_types=[
            pltpu.VMEM((TPW,), jnp.int32),
            pltpu.VMEM((TPW,), jnp.int32),
            pltpu.VMEM((CCH, D), jnp.float32),
            pltpu.VMEM((CCH, D), jnp.float32),
            pltpu.VMEM((CCH, D), jnp.float32),
            pltpu.VMEM((CCH, D), jnp.float32),
            pltpu.SemaphoreType.DMA,
            pltpu.SemaphoreType.DMA,
            pltpu.SemaphoreType.DMA,
            pltpu.SemaphoreType.DMA,
            pltpu.SemaphoreType.DMA,
            pltpu.SemaphoreType.DMA,
        ],
    )


def kernel(hidden_states, gate_w, w_gate, w_up, w_down):
    B, S, d = hidden_states.shape
    x = hidden_states.reshape(-1, d)
    pos0, pos1, tw0, tw1, be = _router(x, gate_w)
    pos_all = jnp.concatenate([pos0, pos1], axis=0).reshape(A // DCH, DCH)
    tw_all = jnp.concatenate([tw0, tw1], axis=0).reshape(A // DCH, DCH)
    xd, ss = _sc_dispatch()(pos_all, tw_all, x)
    y = _ffn(be.reshape(NBLK), xd,
             w_gate.astype(jnp.bfloat16), w_up.astype(jnp.bfloat16),
             w_down.astype(jnp.bfloat16), ss.reshape(PAD, 1))
    out = _sc_combine()(pos0.reshape(NW, TPW), pos1.reshape(NW, TPW), y)
    return out.reshape(B, S, d)


# speedup vs baseline: 1.6231x; 1.0548x over previous
import functools

import jax
import jax.numpy as jnp
from jax import lax
from jax.experimental import pallas as pl
from jax.experimental.pallas import tpu as pltpu
from jax.experimental.pallas import tpu_sc as plsc

E = 8
K = 2
D = 1024
F = 512
T = 4096
A = T * K
BLK = 256
PAD = A + E * BLK
NBLK = PAD // BLK
CHK = 512
NCH = T // CHK

NC, NS = 2, 16
NW = NC * NS
RPW = PAD // NW
TPW = T // NW
GCH = 32
CCH = 16


@functools.lru_cache(maxsize=None)
def _sc_mesh():
    return plsc.VectorSubcoreMesh(
        core_axis_name="c", subcore_axis_name="s",
        num_cores=NC, num_subcores=NS)


def _worker_id():
    return lax.axis_index("s") * NC + lax.axis_index("c")


def _router_body(x_ref, gw_ref, pos0_ref, pos1_ref, tw0_ref, tw1_ref, be_ref,
                 x16_ref):
    x = x_ref[...]
    gw = gw_ref[...]
    logits = lax.dot_general(x, gw, (((1,), (1,)), ((), ())),
                             preferred_element_type=jnp.float32)
    scores = jax.nn.sigmoid(logits)
    ie = lax.broadcasted_iota(jnp.int32, (T, E), 1)
    m1 = jnp.max(scores, axis=1, keepdims=True)
    e0 = jnp.min(jnp.where(scores >= m1, ie, E), axis=1, keepdims=True)
    oh0 = ie == e0
    s2 = jnp.where(oh0, -1.0, scores)
    m2 = jnp.max(s2, axis=1, keepdims=True)
    e1 = jnp.min(jnp.where(s2 >= m2, ie, E), axis=1, keepdims=True)
    oh1 = ie == e1
    den = m1 + m2 + 1e-20
    tw0_ref[...] = m1 / den
    tw1_ref[...] = m2 / den
    u = lax.bitcast_convert_type(x, jnp.uint32)
    ulo = lax.slice_in_dim(u, 0, D // 2, axis=1)
    uhi = lax.slice_in_dim(u, D // 2, D, axis=1)

    def rnd16(v):
        return (v + jnp.uint32(0x7FFF) + ((v >> 16) & jnp.uint32(1))) >> 16

    packed = rnd16(ulo) | (rnd16(uhi) << 16)
    x16_ref[...] = lax.bitcast_convert_type(packed, jnp.float32)

    ind0 = oh0.astype(jnp.float32)
    ind1 = oh1.astype(jnp.float32)
    ri = lax.broadcasted_iota(jnp.int32, (CHK, CHK), 0)
    ci = lax.broadcasted_iota(jnp.int32, (CHK, CHK), 1)
    tstrict = (ci < ri).astype(jnp.float32)

    def chunk_ranks(ind):
        pref = jnp.zeros((1, E), jnp.float32)
        sls, rks = [], []
        for c in range(NCH):
            sl = lax.slice_in_dim(ind, c * CHK, (c + 1) * CHK, axis=0)
            loc = lax.dot_general(tstrict, sl, (((1,), (0,)), ((), ())),
                                  preferred_element_type=jnp.float32)
            rks.append(jnp.sum(sl * (loc + pref), axis=1, keepdims=True))
            sls.append(sl)
            pref = pref + jnp.sum(sl, axis=0, keepdims=True)
        return sls, rks, pref

    sl0, rk0, cnt0 = chunk_ranks(ind0)
    sl1, rk1, cnt1 = chunk_ranks(ind1)
    counts = cnt0 + cnt1
    seg = jnp.floor((counts + (BLK - 1)) * (1.0 / BLK)) * BLK
    ea = lax.broadcasted_iota(jnp.int32, (E, E), 0)
    eb = lax.broadcasted_iota(jnp.int32, (E, E), 1)
    upper = (ea < eb).astype(jnp.float32)
    offs = lax.dot_general(seg, upper, (((1,), (0,)), ((), ())))
    base1 = offs + cnt0
    for c in range(NCH):
        o0 = jnp.sum(sl0[c] * offs, axis=1, keepdims=True)
        pos0_ref[pl.ds(c * CHK, CHK), :] = (o0 + rk0[c]).astype(jnp.int32)
        o1 = jnp.sum(sl1[c] * base1, axis=1, keepdims=True)
        pos1_ref[pl.ds(c * CHK, CHK), :] = (o1 + rk1[c]).astype(jnp.int32)
    ends = offs + seg
    bi = (lax.broadcasted_iota(jnp.int32, (NBLK, 1), 0) * BLK
          ).astype(jnp.float32)
    be = jnp.sum((bi >= ends).astype(jnp.float32), axis=1, keepdims=True)
    be_ref[...] = jnp.minimum(be, E - 1.0).astype(jnp.int32)


def _router(x, gate_w):
    return pl.pallas_call(
        _router_body,
        out_shape=[
            jax.ShapeDtypeStruct((T, 1), jnp.int32),
            jax.ShapeDtypeStruct((T, 1), jnp.int32),
            jax.ShapeDtypeStruct((T, 1), jnp.float32),
            jax.ShapeDtypeStruct((T, 1), jnp.float32),
            jax.ShapeDtypeStruct((NBLK, 1), jnp.int32),
            jax.ShapeDtypeStruct((T, D // 2), jnp.float32),
        ],
    )(x, gate_w)


DCH = 64
NCHD = (A // NW) // DCH


def _sc_dispatch_body(pos3_ref, tw3_ref, x_ref, xd_ref, ss_ref,
                      pvw, tww, buf0, buf1, gs0, gs1, ws0, ws1, ssem):
    wid = _worker_id()
    row0 = wid * NCHD
    pltpu.sync_copy(pos3_ref.at[pl.ds(row0, NCHD)], pvw)
    pltpu.sync_copy(tw3_ref.at[pl.ds(row0, NCHD)], tww)
    sdescs = []
    for j in range(NCHD):
        sdescs.append(
            pltpu.async_copy(tww.at[j], ss_ref.at[pvw.at[j]], ssem))
    t0 = (wid * (A // NW)) & (T - 1)

    bufs, gsems, wsems = (buf0, buf1), (gs0, gs1), (ws0, ws1)
    gd = [None, None]
    wd = [None, None]

    def start(c):
        b = c & 1
        gd[b] = pltpu.async_copy(
            x_ref.at[pl.ds(pl.multiple_of(t0 + c * DCH, 8), DCH)],
            bufs[b], gsems[b])

    start(0)
    for c in range(NCHD):
        b = c & 1
        gd[b].wait()
        if c + 1 < NCHD:
            if wd[1 - b] is not None:
                wd[1 - b].wait()
            start(c + 1)
        wd[b] = pltpu.async_copy(bufs[b], xd_ref.at[pvw.at[c]], wsems[b])
    wd[0].wait()
    wd[1].wait()
    for sd in sdescs:
        sd.wait()


@functools.lru_cache(maxsize=None)
def _sc_dispatch():
    return pl.kernel(
        _sc_dispatch_body,
        out_type=[
            jax.ShapeDtypeStruct((PAD, D // 2), jnp.float32),
            jax.ShapeDtypeStruct((PAD,), jnp.float32),
        ],
        mesh=_sc_mesh(),
        scratch_types=[
            pltpu.VMEM((NCHD, DCH), jnp.int32),
            pltpu.VMEM((NCHD, DCH), jnp.float32),
            pltpu.VMEM((DCH, D // 2), jnp.float32),
            pltpu.VMEM((DCH, D // 2), jnp.float32),
            pltpu.SemaphoreType.DMA,
            pltpu.SemaphoreType.DMA,
            pltpu.SemaphoreType.DMA,
            pltpu.SemaphoreType.DMA,
            pltpu.SemaphoreType.DMA,
        ],
    )


def _ffn_body(be_ref, xd_ref, wg_ref, wu_ref, wd_ref, ss_ref, y_ref):
    p = lax.bitcast_convert_type(xd_ref[...], jnp.uint32)
    xlo = lax.bitcast_convert_type(p << 16, jnp.float32)
    xhi = lax.bitcast_convert_type(p & jnp.uint32(0xFFFF0000), jnp.float32)
    xb = lax.concatenate(
        [xlo.astype(jnp.bfloat16), xhi.astype(jnp.bfloat16)], 1)
    g = lax.dot_general(xb, wg_ref[0], (((1,), (1,)), ((), ())),
                        preferred_element_type=jnp.float32)
    u = lax.dot_general(xb, wu_ref[0], (((1,), (1,)), ((), ())),
                        preferred_element_type=jnp.float32)
    h = (g * jax.nn.sigmoid(g) * u).astype(jnp.bfloat16)
    y = lax.dot_general(h, wd_ref[0], (((1,), (1,)), ((), ())),
                        preferred_element_type=jnp.float32)
    y_ref[...] = y * ss_ref[...]


def _ffn(be, xd, w_gate, w_up, w_down, ss):
    grid_spec = pltpu.PrefetchScalarGridSpec(
        num_scalar_prefetch=1,
        grid=(NBLK,),
        in_specs=[
            pl.BlockSpec((BLK, D // 2), lambda i, be: (i, 0)),
            pl.BlockSpec((1, F, D), lambda i, be: (be[i], 0, 0)),
            pl.BlockSpec((1, F, D), lambda i, be: (be[i], 0, 0)),
            pl.BlockSpec((1, D, F), lambda i, be: (be[i], 0, 0)),
            pl.BlockSpec((BLK, 1), lambda i, be: (i, 0)),
        ],
        out_specs=pl.BlockSpec((BLK, D), lambda i, be: (i, 0)),
    )
    return pl.pallas_call(
        _ffn_body,
        grid_spec=grid_spec,
        out_shape=jax.ShapeDtypeStruct((PAD, D), jnp.float32),
        compiler_params=pltpu.CompilerParams(
            dimension_semantics=("arbitrary",),
        ),
    )(be, xd, w_gate, w_up, w_down, ss)


def _sc_combine_body(p02_ref, p12_ref, y_ref, out_ref, p0w, p1w,
                     ya0, yb0, ya1, yb1, ga0, gb0, ga1, gb1, ws0, ws1):
    wid = _worker_id()
    tok0 = pl.multiple_of(wid * TPW, 8)
    pltpu.sync_copy(p02_ref.at[wid], p0w)
    pltpu.sync_copy(p12_ref.at[wid], p1w)
    yas, ybs = (ya0, ya1), (yb0, yb1)
    gas, gbs, wss = (ga0, ga1), (gb0, gb1), (ws0, ws1)
    gda = [None, None]
    gdb = [None, None]
    wd = [None, None]
    NCC = TPW // CCH

    def start(c):
        b = c & 1
        ia = p0w.at[pl.ds(c * CCH, CCH)]
        ib = p1w.at[pl.ds(c * CCH, CCH)]
        gda[b] = pltpu.async_copy(y_ref.at[ia], yas[b], gas[b])
        gdb[b] = pltpu.async_copy(y_ref.at[ib], ybs[b], gbs[b])

    start(0)
    for c in range(NCC):
        b = c & 1
        gda[b].wait()
        gdb[b].wait()
        if c + 1 < NCC:
            if wd[1 - b] is not None:
                wd[1 - b].wait()
            start(c + 1)
        ya, yb = yas[b], ybs[b]

        def add(j, carry):
            off = pl.multiple_of(j * 16, 16)
            for r in range(CCH):
                ya[r, pl.ds(off, 16)] = (ya[r, pl.ds(off, 16)]
                                         + yb[r, pl.ds(off, 16)])
            return carry

        lax.fori_loop(0, D // 16, add, 0)
        wd[b] = pltpu.async_copy(
            ya, out_ref.at[pl.ds(tok0 + c * CCH, CCH)], wss[b])
    wd[0].wait()
    wd[1].wait()


@functools.lru_cache(maxsize=None)
def _sc_combine():
    return pl.kernel(
        _sc_combine_body,
        out_type=jax.ShapeDtypeStruct((T, D), jnp.float32),
        mesh=_sc_mesh(),
        scratch_types=[
            pltpu.VMEM((TPW,), jnp.int32),
            pltpu.VMEM((TPW,), jnp.int32),
            pltpu.VMEM((CCH, D), jnp.float32),
            pltpu.VMEM((CCH, D), jnp.float32),
            pltpu.VMEM((CCH, D), jnp.float32),
            pltpu.VMEM((CCH, D), jnp.float32),
            pltpu.SemaphoreType.DMA,
            pltpu.SemaphoreType.DMA,
            pltpu.SemaphoreType.DMA,
            pltpu.SemaphoreType.DMA,
            pltpu.SemaphoreType.DMA,
            pltpu.SemaphoreType.DMA,
        ],
    )


def kernel(hidden_states, gate_w, w_gate, w_up, w_down):
    B, S, d = hidden_states.shape
    x = hidden_states.reshape(-1, d)
    pos0, pos1, tw0, tw1, be, x16 = _router(x, gate_w)
    pos_all = jnp.concatenate([pos0, pos1], axis=0).reshape(A // DCH, DCH)
    tw_all = jnp.concatenate([tw0, tw1], axis=0).reshape(A // DCH, DCH)
    xd, ss = _sc_dispatch()(pos_all, tw_all, x16)
    y = _ffn(be.reshape(NBLK), xd,
             w_gate.astype(jnp.bfloat16), w_up.astype(jnp.bfloat16),
             w_down.astype(jnp.bfloat16), ss.reshape(PAD, 1))
    out = _sc_combine()(pos0.reshape(NW, TPW), pos1.reshape(NW, TPW), y)
    return out.reshape(B, S, d)


# speedup vs baseline: 1.7297x; 1.0657x over previous
import functools

import jax
import jax.numpy as jnp
from jax import lax
from jax.experimental import pallas as pl
from jax.experimental.pallas import tpu as pltpu
from jax.experimental.pallas import tpu_sc as plsc

E = 8
K = 2
D = 1024
F = 512
T = 4096
A = T * K
BLK = 512
PAD = A + E * BLK
NBLK = PAD // BLK
CHK = 512
NCH = T // CHK

NC, NS = 2, 16
NW = NC * NS
RPW = PAD // NW
TPW = T // NW
GCH = 32
CCH = 16


@functools.lru_cache(maxsize=None)
def _sc_mesh():
    return plsc.VectorSubcoreMesh(
        core_axis_name="c", subcore_axis_name="s",
        num_cores=NC, num_subcores=NS)


def _worker_id():
    return lax.axis_index("s") * NC + lax.axis_index("c")


def _router_body(x_ref, gw_ref, pos_ref, tw_ref, be_ref, x16_ref):
    x = x_ref[...]
    gw = gw_ref[...]
    logits = lax.dot_general(x, gw, (((1,), (1,)), ((), ())),
                             preferred_element_type=jnp.float32)
    scores = jax.nn.sigmoid(logits)
    ie = lax.broadcasted_iota(jnp.int32, (T, E), 1)
    m1 = jnp.max(scores, axis=1, keepdims=True)
    e0 = jnp.min(jnp.where(scores >= m1, ie, E), axis=1, keepdims=True)
    oh0 = ie == e0
    s2 = jnp.where(oh0, -1.0, scores)
    m2 = jnp.max(s2, axis=1, keepdims=True)
    e1 = jnp.min(jnp.where(s2 >= m2, ie, E), axis=1, keepdims=True)
    oh1 = ie == e1
    den = m1 + m2 + 1e-20
    tw_ref[pl.ds(0, T), :] = m1 / den
    tw_ref[pl.ds(T, T), :] = m2 / den
    u = lax.bitcast_convert_type(x, jnp.uint32)
    ulo = lax.slice_in_dim(u, 0, D // 2, axis=1)
    uhi = lax.slice_in_dim(u, D // 2, D, axis=1)

    def rnd16(v):
        return (v + jnp.uint32(0x7FFF) + ((v >> 16) & jnp.uint32(1))) >> 16

    packed = rnd16(ulo) | (rnd16(uhi) << 16)
    x16_ref[...] = lax.bitcast_convert_type(packed, jnp.float32)

    ind0 = oh0.astype(jnp.float32)
    ind1 = oh1.astype(jnp.float32)
    ri = lax.broadcasted_iota(jnp.int32, (CHK, CHK), 0)
    ci = lax.broadcasted_iota(jnp.int32, (CHK, CHK), 1)
    tstrict = (ci < ri).astype(jnp.float32)

    def chunk_ranks(ind):
        pref = jnp.zeros((1, E), jnp.float32)
        sls, rks = [], []
        for c in range(NCH):
            sl = lax.slice_in_dim(ind, c * CHK, (c + 1) * CHK, axis=0)
            loc = lax.dot_general(tstrict, sl, (((1,), (0,)), ((), ())),
                                  preferred_element_type=jnp.float32)
            rks.append(jnp.sum(sl * (loc + pref), axis=1, keepdims=True))
            sls.append(sl)
            pref = pref + jnp.sum(sl, axis=0, keepdims=True)
        return sls, rks, pref

    sl0, rk0, cnt0 = chunk_ranks(ind0)
    sl1, rk1, cnt1 = chunk_ranks(ind1)
    counts = cnt0 + cnt1
    seg = jnp.floor((counts + (BLK - 1)) * (1.0 / BLK)) * BLK
    ea = lax.broadcasted_iota(jnp.int32, (E, E), 0)
    eb = lax.broadcasted_iota(jnp.int32, (E, E), 1)
    upper = (ea < eb).astype(jnp.float32)
    offs = lax.dot_general(seg, upper, (((1,), (0,)), ((), ())))
    base1 = offs + cnt0
    for c in range(NCH):
        o0 = jnp.sum(sl0[c] * offs, axis=1, keepdims=True)
        pos_ref[pl.ds(c * CHK, CHK), :] = (o0 + rk0[c]).astype(jnp.int32)
        o1 = jnp.sum(sl1[c] * base1, axis=1, keepdims=True)
        pos_ref[pl.ds(T + c * CHK, CHK), :] = (o1 + rk1[c]).astype(jnp.int32)
    ends = offs + seg
    bi = (lax.broadcasted_iota(jnp.int32, (NBLK, 1), 0) * BLK
          ).astype(jnp.float32)
    be = jnp.sum((bi >= ends).astype(jnp.float32), axis=1, keepdims=True)
    be_ref[...] = jnp.minimum(be, E - 1.0).astype(jnp.int32)


def _router(x, gate_w):
    return pl.pallas_call(
        _router_body,
        out_shape=[
            jax.ShapeDtypeStruct((A, 1), jnp.int32),
            jax.ShapeDtypeStruct((A, 1), jnp.float32),
            jax.ShapeDtypeStruct((NBLK, 1), jnp.int32),
            jax.ShapeDtypeStruct((T, D // 2), jnp.float32),
        ],
    )(x, gate_w)


DCH = 64
NCHD = (A // NW) // DCH


def _sc_dispatch_body(pos3_ref, tw3_ref, x_ref, xd_ref, ss_ref,
                      pvw, tww, buf0, buf1, gs0, gs1, ws0, ws1, ssem):
    wid = _worker_id()
    row0 = wid * NCHD
    pltpu.sync_copy(pos3_ref.at[pl.ds(row0, NCHD)], pvw)
    pltpu.sync_copy(tw3_ref.at[pl.ds(row0, NCHD)], tww)
    sdescs = []
    for j in range(NCHD):
        sdescs.append(
            pltpu.async_copy(tww.at[j], ss_ref.at[pvw.at[j]], ssem))
    t0 = (wid * (A // NW)) & (T - 1)

    bufs, gsems, wsems = (buf0, buf1), (gs0, gs1), (ws0, ws1)
    gd = [None, None]
    wd = [None, None]

    def start(c):
        b = c & 1
        gd[b] = pltpu.async_copy(
            x_ref.at[pl.ds(pl.multiple_of(t0 + c * DCH, 8), DCH)],
            bufs[b], gsems[b])

    start(0)
    for c in range(NCHD):
        b = c & 1
        gd[b].wait()
        if c + 1 < NCHD:
            if wd[1 - b] is not None:
                wd[1 - b].wait()
            start(c + 1)
        wd[b] = pltpu.async_copy(bufs[b], xd_ref.at[pvw.at[c]], wsems[b])
    wd[0].wait()
    wd[1].wait()
    for sd in sdescs:
        sd.wait()


@functools.lru_cache(maxsize=None)
def _sc_dispatch():
    return pl.kernel(
        _sc_dispatch_body,
        out_type=[
            jax.ShapeDtypeStruct((PAD, D // 2), jnp.float32),
            jax.ShapeDtypeStruct((PAD,), jnp.float32),
        ],
        mesh=_sc_mesh(),
        scratch_types=[
            pltpu.VMEM((NCHD, DCH), jnp.int32),
            pltpu.VMEM((NCHD, DCH), jnp.float32),
            pltpu.VMEM((DCH, D // 2), jnp.float32),
            pltpu.VMEM((DCH, D // 2), jnp.float32),
            pltpu.SemaphoreType.DMA,
            pltpu.SemaphoreType.DMA,
            pltpu.SemaphoreType.DMA,
            pltpu.SemaphoreType.DMA,
            pltpu.SemaphoreType.DMA,
        ],
    )


def _ffn_body(be_ref, xd_ref, wg_ref, wu_ref, wd_ref, ss_ref, y_ref):
    p = lax.bitcast_convert_type(xd_ref[...], jnp.uint32)
    xlo = lax.bitcast_convert_type(p << 16, jnp.float32)
    xhi = lax.bitcast_convert_type(p & jnp.uint32(0xFFFF0000), jnp.float32)
    xb = lax.concatenate(
        [xlo.astype(jnp.bfloat16), xhi.astype(jnp.bfloat16)], 1)
    g = lax.dot_general(xb, wg_ref[0], (((1,), (1,)), ((), ())),
                        preferred_element_type=jnp.float32)
    u = lax.dot_general(xb, wu_ref[0], (((1,), (1,)), ((), ())),
                        preferred_element_type=jnp.float32)
    h = (g * jax.nn.sigmoid(g) * u).astype(jnp.bfloat16)
    y = lax.dot_general(h, wd_ref[0], (((1,), (1,)), ((), ())),
                        preferred_element_type=jnp.float32)
    y_ref[...] = y * ss_ref[...].reshape(BLK, 1)


def _ffn(be, xd, w_gate, w_up, w_down, ss):
    grid_spec = pltpu.PrefetchScalarGridSpec(
        num_scalar_prefetch=1,
        grid=(NBLK,),
        in_specs=[
            pl.BlockSpec((BLK, D // 2), lambda i, be: (i, 0)),
            pl.BlockSpec((1, F, D), lambda i, be: (be[i], 0, 0)),
            pl.BlockSpec((1, F, D), lambda i, be: (be[i], 0, 0)),
            pl.BlockSpec((1, D, F), lambda i, be: (be[i], 0, 0)),
            pl.BlockSpec((BLK,), lambda i, be: (i,)),
        ],
        out_specs=pl.BlockSpec((BLK, D), lambda i, be: (i, 0)),
    )
    return pl.pallas_call(
        _ffn_body,
        grid_spec=grid_spec,
        out_shape=jax.ShapeDtypeStruct((PAD, D), jnp.float32),
        compiler_params=pltpu.CompilerParams(
            dimension_semantics=("arbitrary",),
        ),
    )(be, xd, w_gate, w_up, w_down, ss)


def _sc_combine_body(p02_ref, p12_ref, y_ref, out_ref, p0w, p1w,
                     ya0, yb0, ya1, yb1, ga0, gb0, ga1, gb1, ws0, ws1):
    wid = _worker_id()
    tok0 = pl.multiple_of(wid * TPW, 8)
    pltpu.sync_copy(p02_ref.at[wid], p0w)
    pltpu.sync_copy(p12_ref.at[wid], p1w)
    yas, ybs = (ya0, ya1), (yb0, yb1)
    gas, gbs, wss = (ga0, ga1), (gb0, gb1), (ws0, ws1)
    gda = [None, None]
    gdb = [None, None]
    wd = [None, None]
    NCC = TPW // CCH

    def start(c):
        b = c & 1
        ia = p0w.at[pl.ds(c * CCH, CCH)]
        ib = p1w.at[pl.ds(c * CCH, CCH)]
        gda[b] = pltpu.async_copy(y_ref.at[ia], yas[b], gas[b])
        gdb[b] = pltpu.async_copy(y_ref.at[ib], ybs[b], gbs[b])

    start(0)
    for c in range(NCC):
        b = c & 1
        gda[b].wait()
        gdb[b].wait()
        if c + 1 < NCC:
            if wd[1 - b] is not None:
                wd[1 - b].wait()
            start(c + 1)
        ya, yb = yas[b], ybs[b]

        def add(j, carry):
            off = pl.multiple_of(j * 16, 16)
            for r in range(CCH):
                ya[r, pl.ds(off, 16)] = (ya[r, pl.ds(off, 16)]
                                         + yb[r, pl.ds(off, 16)])
            return carry

        lax.fori_loop(0, D // 16, add, 0)
        wd[b] = pltpu.async_copy(
            ya, out_ref.at[pl.ds(tok0 + c * CCH, CCH)], wss[b])
    wd[0].wait()
    wd[1].wait()


@functools.lru_cache(maxsize=None)
def _sc_combine():
    return pl.kernel(
        _sc_combine_body,
        out_type=jax.ShapeDtypeStruct((T, D), jnp.float32),
        mesh=_sc_mesh(),
        scratch_types=[
            pltpu.VMEM((TPW,), jnp.int32),
            pltpu.VMEM((TPW,), jnp.int32),
            pltpu.VMEM((CCH, D), jnp.float32),
            pltpu.VMEM((CCH, D), jnp.float32),
            pltpu.VMEM((CCH, D), jnp.float32),
            pltpu.VMEM((CCH, D), jnp.float32),
            pltpu.SemaphoreType.DMA,
            pltpu.SemaphoreType.DMA,
            pltpu.SemaphoreType.DMA,
            pltpu.SemaphoreType.DMA,
            pltpu.SemaphoreType.DMA,
            pltpu.SemaphoreType.DMA,
        ],
    )


def kernel(hidden_states, gate_w, w_gate, w_up, w_down):
    B, S, d = hidden_states.shape
    x = hidden_states.reshape(-1, d)
    pos_all, tw_all, be, x16 = _router(x, gate_w)
    xd, ss = _sc_dispatch()(pos_all.reshape(A // DCH, DCH),
                            tw_all.reshape(A // DCH, DCH), x16)
    y = _ffn(be.reshape(NBLK), xd,
             w_gate.astype(jnp.bfloat16), w_up.astype(jnp.bfloat16),
             w_down.astype(jnp.bfloat16), ss)
    pos2 = pos_all.reshape(K, NW, TPW)
    out = _sc_combine()(pos2[0], pos2[1], y)
    return out.reshape(B, S, d)


# speedup vs baseline: 1.7394x; 1.0056x over previous
import functools

import jax
import jax.numpy as jnp
from jax import lax
from jax.experimental import pallas as pl
from jax.experimental.pallas import tpu as pltpu
from jax.experimental.pallas import tpu_sc as plsc

E = 8
K = 2
D = 1024
F = 512
T = 4096
A = T * K
BLK = 512
PAD = A + E * BLK
NBLK = PAD // BLK
CHK = 512
NCH = T // CHK

NC, NS = 2, 16
NW = NC * NS
RPW = PAD // NW
TPW = T // NW
GCH = 32
CCH = 16


@functools.lru_cache(maxsize=None)
def _sc_mesh():
    return plsc.VectorSubcoreMesh(
        core_axis_name="c", subcore_axis_name="s",
        num_cores=NC, num_subcores=NS)


def _worker_id():
    return lax.axis_index("s") * NC + lax.axis_index("c")


def _router_body(x_ref, gw_ref, pos_ref, tw_ref, be_ref, x16_ref):
    x = x_ref[...]
    gw = gw_ref[...]
    logits = lax.dot_general(x, gw, (((1,), (1,)), ((), ())),
                             preferred_element_type=jnp.float32)
    scores = jax.nn.sigmoid(logits)
    ie = lax.broadcasted_iota(jnp.int32, (T, E), 1)
    m1 = jnp.max(scores, axis=1, keepdims=True)
    e0 = jnp.min(jnp.where(scores >= m1, ie, E), axis=1, keepdims=True)
    oh0 = ie == e0
    s2 = jnp.where(oh0, -1.0, scores)
    m2 = jnp.max(s2, axis=1, keepdims=True)
    e1 = jnp.min(jnp.where(s2 >= m2, ie, E), axis=1, keepdims=True)
    oh1 = ie == e1
    den = m1 + m2 + 1e-20
    tw_ref[pl.ds(0, T), :] = m1 / den
    tw_ref[pl.ds(T, T), :] = m2 / den
    u = lax.bitcast_convert_type(x, jnp.uint32)
    ulo = lax.slice_in_dim(u, 0, D // 2, axis=1)
    uhi = lax.slice_in_dim(u, D // 2, D, axis=1)

    def rnd16(v):
        return (v + jnp.uint32(0x7FFF) + ((v >> 16) & jnp.uint32(1))) >> 16

    packed = rnd16(ulo) | (rnd16(uhi) << 16)
    x16_ref[...] = lax.bitcast_convert_type(packed, jnp.float32)

    ind0 = oh0.astype(jnp.float32)
    ind1 = oh1.astype(jnp.float32)
    ri = lax.broadcasted_iota(jnp.int32, (CHK, CHK), 0)
    ci = lax.broadcasted_iota(jnp.int32, (CHK, CHK), 1)
    tstrict = (ci < ri).astype(jnp.float32)

    def chunk_ranks(ind):
        pref = jnp.zeros((1, E), jnp.float32)
        sls, rks = [], []
        for c in range(NCH):
            sl = lax.slice_in_dim(ind, c * CHK, (c + 1) * CHK, axis=0)
            loc = lax.dot_general(tstrict, sl, (((1,), (0,)), ((), ())),
                                  preferred_element_type=jnp.float32)
            rks.append(jnp.sum(sl * (loc + pref), axis=1, keepdims=True))
            sls.append(sl)
            pref = pref + jnp.sum(sl, axis=0, keepdims=True)
        return sls, rks, pref

    sl0, rk0, cnt0 = chunk_ranks(ind0)
    sl1, rk1, cnt1 = chunk_ranks(ind1)
    counts = cnt0 + cnt1
    seg = jnp.floor((counts + (BLK - 1)) * (1.0 / BLK)) * BLK
    ea = lax.broadcasted_iota(jnp.int32, (E, E), 0)
    eb = lax.broadcasted_iota(jnp.int32, (E, E), 1)
    upper = (ea < eb).astype(jnp.float32)
    offs = lax.dot_general(seg, upper, (((1,), (0,)), ((), ())))
    base1 = offs + cnt0
    for c in range(NCH):
        o0 = jnp.sum(sl0[c] * offs, axis=1, keepdims=True)
        pos_ref[pl.ds(c * CHK, CHK), :] = (o0 + rk0[c]).astype(jnp.int32)
        o1 = jnp.sum(sl1[c] * base1, axis=1, keepdims=True)
        pos_ref[pl.ds(T + c * CHK, CHK), :] = (o1 + rk1[c]).astype(jnp.int32)
    ends = offs + seg
    bi = (lax.broadcasted_iota(jnp.int32, (NBLK, 1), 0) * BLK
          ).astype(jnp.float32)
    be = jnp.sum((bi >= ends).astype(jnp.float32), axis=1, keepdims=True)
    be_ref[...] = jnp.minimum(be, E - 1.0).astype(jnp.int32)


def _router(x, gate_w):
    return pl.pallas_call(
        _router_body,
        out_shape=[
            jax.ShapeDtypeStruct((A, 1), jnp.int32),
            jax.ShapeDtypeStruct((A, 1), jnp.float32),
            jax.ShapeDtypeStruct((NBLK, 1), jnp.int32),
            jax.ShapeDtypeStruct((T, D // 2), jnp.float32),
        ],
    )(x, gate_w)


DCH = 64
NCHD = (A // NW) // DCH


def _sc_dispatch_body(pos3_ref, tw3_ref, x_ref, xd_ref, ss_ref,
                      pvw, tww, buf0, buf1, buf2,
                      gs0, gs1, gs2, ws0, ws1, ws2, ssem):
    wid = _worker_id()
    row0 = wid * NCHD
    pltpu.sync_copy(pos3_ref.at[pl.ds(row0, NCHD)], pvw)
    pltpu.sync_copy(tw3_ref.at[pl.ds(row0, NCHD)], tww)
    sdescs = []
    for j in range(NCHD):
        sdescs.append(
            pltpu.async_copy(tww.at[j], ss_ref.at[pvw.at[j]], ssem))
    t0 = (wid * (A // NW)) & (T - 1)

    bufs, gsems, wsems = (buf0, buf1, buf2), (gs0, gs1, gs2), (ws0, ws1, ws2)
    gd = [None, None, None]
    wd = [None, None, None]

    def start(c):
        b = c % 3
        gd[b] = pltpu.async_copy(
            x_ref.at[pl.ds(pl.multiple_of(t0 + c * DCH, 8), DCH)],
            bufs[b], gsems[b])

    start(0)
    start(1)
    for c in range(NCHD):
        b = c % 3
        gd[b].wait()
        if c + 2 < NCHD:
            nb = (c + 2) % 3
            if wd[nb] is not None:
                wd[nb].wait()
            start(c + 2)
        wd[b] = pltpu.async_copy(bufs[b], xd_ref.at[pvw.at[c]], wsems[b])
    for b in range(3):
        if wd[b] is not None:
            wd[b].wait()
    for sd in sdescs:
        sd.wait()


@functools.lru_cache(maxsize=None)
def _sc_dispatch():
    return pl.kernel(
        _sc_dispatch_body,
        out_type=[
            jax.ShapeDtypeStruct((PAD, D // 2), jnp.float32),
            jax.ShapeDtypeStruct((PAD,), jnp.float32),
        ],
        mesh=_sc_mesh(),
        scratch_types=[
            pltpu.VMEM((NCHD, DCH), jnp.int32),
            pltpu.VMEM((NCHD, DCH), jnp.float32),
            pltpu.VMEM((DCH, D // 2), jnp.float32),
            pltpu.VMEM((DCH, D // 2), jnp.float32),
            pltpu.VMEM((DCH, D // 2), jnp.float32),
            pltpu.SemaphoreType.DMA,
            pltpu.SemaphoreType.DMA,
            pltpu.SemaphoreType.DMA,
            pltpu.SemaphoreType.DMA,
            pltpu.SemaphoreType.DMA,
            pltpu.SemaphoreType.DMA,
            pltpu.SemaphoreType.DMA,
        ],
    )


def _ffn_body(be_ref, xd_ref, wg_ref, wu_ref, wd_ref, ss_ref, y_ref):
    p = lax.bitcast_convert_type(xd_ref[...], jnp.uint32)
    xlo = lax.bitcast_convert_type(p << 16, jnp.float32)
    xhi = lax.bitcast_convert_type(p & jnp.uint32(0xFFFF0000), jnp.float32)
    xb = lax.concatenate(
        [xlo.astype(jnp.bfloat16), xhi.astype(jnp.bfloat16)], 1)
    g = lax.dot_general(xb, wg_ref[0], (((1,), (1,)), ((), ())),
                        preferred_element_type=jnp.float32)
    u = lax.dot_general(xb, wu_ref[0], (((1,), (1,)), ((), ())),
                        preferred_element_type=jnp.float32)
    h = (g * jax.nn.sigmoid(g) * u).astype(jnp.bfloat16)
    y = lax.dot_general(h, wd_ref[0], (((1,), (1,)), ((), ())),
                        preferred_element_type=jnp.float32)
    y_ref[...] = y * ss_ref[...].reshape(BLK, 1)


def _ffn(be, xd, w_gate, w_up, w_down, ss):
    grid_spec = pltpu.PrefetchScalarGridSpec(
        num_scalar_prefetch=1,
        grid=(NBLK,),
        in_specs=[
            pl.BlockSpec((BLK, D // 2), lambda i, be: (i, 0)),
            pl.BlockSpec((1, F, D), lambda i, be: (be[i], 0, 0)),
            pl.BlockSpec((1, F, D), lambda i, be: (be[i], 0, 0)),
            pl.BlockSpec((1, D, F), lambda i, be: (be[i], 0, 0)),
            pl.BlockSpec((BLK,), lambda i, be: (i,)),
        ],
        out_specs=pl.BlockSpec((BLK, D), lambda i, be: (i, 0)),
    )
    return pl.pallas_call(
        _ffn_body,
        grid_spec=grid_spec,
        out_shape=jax.ShapeDtypeStruct((PAD, D), jnp.float32),
        compiler_params=pltpu.CompilerParams(
            dimension_semantics=("arbitrary",),
        ),
    )(be, xd, w_gate, w_up, w_down, ss)


def _sc_combine_body(p02_ref, p12_ref, y_ref, out_ref, p0w, p1w,
                     ya0, yb0, ya1, yb1, ga0, gb0, ga1, gb1, ws0, ws1):
    wid = _worker_id()
    tok0 = pl.multiple_of(wid * TPW, 8)
    pltpu.sync_copy(p02_ref.at[wid], p0w)
    pltpu.sync_copy(p12_ref.at[wid], p1w)
    yas, ybs = (ya0, ya1), (yb0, yb1)
    gas, gbs, wss = (ga0, ga1), (gb0, gb1), (ws0, ws1)
    gda = [None, None]
    gdb = [None, None]
    wd = [None, None]
    NCC = TPW // CCH

    def start(c):
        b = c & 1
        ia = p0w.at[pl.ds(c * CCH, CCH)]
        ib = p1w.at[pl.ds(c * CCH, CCH)]
        gda[b] = pltpu.async_copy(y_ref.at[ia], yas[b], gas[b])
        gdb[b] = pltpu.async_copy(y_ref.at[ib], ybs[b], gbs[b])

    start(0)
    for c in range(NCC):
        b = c & 1
        gda[b].wait()
        gdb[b].wait()
        if c + 1 < NCC:
            if wd[1 - b] is not None:
                wd[1 - b].wait()
            start(c + 1)
        ya, yb = yas[b], ybs[b]

        def add(j, carry):
            off = pl.multiple_of(j * 16, 16)
            for r in range(CCH):
                ya[r, pl.ds(off, 16)] = (ya[r, pl.ds(off, 16)]
                                         + yb[r, pl.ds(off, 16)])
            return carry

        lax.fori_loop(0, D // 16, add, 0)
        wd[b] = pltpu.async_copy(
            ya, out_ref.at[pl.ds(tok0 + c * CCH, CCH)], wss[b])
    wd[0].wait()
    wd[1].wait()


@functools.lru_cache(maxsize=None)
def _sc_combine():
    return pl.kernel(
        _sc_combine_body,
        out_type=jax.ShapeDtypeStruct((T, D), jnp.float32),
        mesh=_sc_mesh(),
        scratch_types=[
            pltpu.VMEM((TPW,), jnp.int32),
            pltpu.VMEM((TPW,), jnp.int32),
            pltpu.VMEM((CCH, D), jnp.float32),
            pltpu.VMEM((CCH, D), jnp.float32),
            pltpu.VMEM((CCH, D), jnp.float32),
            pltpu.VMEM((CCH, D), jnp.float32),
            pltpu.SemaphoreType.DMA,
            pltpu.SemaphoreType.DMA,
            pltpu.SemaphoreType.DMA,
            pltpu.SemaphoreType.DMA,
            pltpu.SemaphoreType.DMA,
            pltpu.SemaphoreType.DMA,
        ],
    )


def kernel(hidden_states, gate_w, w_gate, w_up, w_down):
    B, S, d = hidden_states.shape
    x = hidden_states.reshape(-1, d)
    pos_all, tw_all, be, x16 = _router(x, gate_w)
    xd, ss = _sc_dispatch()(pos_all.reshape(A // DCH, DCH),
                            tw_all.reshape(A // DCH, DCH), x16)
    y = _ffn(be.reshape(NBLK), xd,
             w_gate.astype(jnp.bfloat16), w_up.astype(jnp.bfloat16),
             w_down.astype(jnp.bfloat16), ss)
    pos2 = pos_all.reshape(K, NW, TPW)
    out = _sc_combine()(pos2[0], pos2[1], y)
    return out.reshape(B, S, d)


# speedup vs baseline: 1.7601x; 1.0119x over previous
import functools

import jax
import jax.numpy as jnp
from jax import lax
from jax.experimental import pallas as pl
from jax.experimental.pallas import tpu as pltpu
from jax.experimental.pallas import tpu_sc as plsc

E = 8
K = 2
D = 1024
F = 512
T = 4096
A = T * K
BLK = 512
PAD = A + E * BLK
NBLK = PAD // BLK
CHK = 512
NCH = T // CHK

NC, NS = 2, 16
NW = NC * NS
RPW = PAD // NW
TPW = T // NW
GCH = 32
CCH = 16


@functools.lru_cache(maxsize=None)
def _sc_mesh():
    return plsc.VectorSubcoreMesh(
        core_axis_name="c", subcore_axis_name="s",
        num_cores=NC, num_subcores=NS)


def _worker_id():
    return lax.axis_index("s") * NC + lax.axis_index("c")


def _router_body(x_ref, gw_ref, pos_ref, tw_ref, be_ref, x16_ref):
    x = x_ref[...]
    gw = gw_ref[...]
    logits = lax.dot_general(x, gw, (((1,), (1,)), ((), ())),
                             preferred_element_type=jnp.float32)
    scores = jax.nn.sigmoid(logits)
    ie = lax.broadcasted_iota(jnp.int32, (T, E), 1)
    m1 = jnp.max(scores, axis=1, keepdims=True)
    e0 = jnp.min(jnp.where(scores >= m1, ie, E), axis=1, keepdims=True)
    oh0 = ie == e0
    s2 = jnp.where(oh0, -1.0, scores)
    m2 = jnp.max(s2, axis=1, keepdims=True)
    e1 = jnp.min(jnp.where(s2 >= m2, ie, E), axis=1, keepdims=True)
    oh1 = ie == e1
    den = m1 + m2 + 1e-20
    tw_ref[pl.ds(0, T), :] = m1 / den
    tw_ref[pl.ds(T, T), :] = m2 / den
    u = lax.bitcast_convert_type(x, jnp.uint32)
    ulo = lax.slice_in_dim(u, 0, D // 2, axis=1)
    uhi = lax.slice_in_dim(u, D // 2, D, axis=1)

    def rnd16(v):
        return (v + jnp.uint32(0x7FFF) + ((v >> 16) & jnp.uint32(1))) >> 16

    packed = rnd16(ulo) | (rnd16(uhi) << 16)
    x16_ref[...] = lax.bitcast_convert_type(packed, jnp.float32)

    ind0 = oh0.astype(jnp.float32)
    ind1 = oh1.astype(jnp.float32)
    ri = lax.broadcasted_iota(jnp.int32, (CHK, CHK), 0)
    ci = lax.broadcasted_iota(jnp.int32, (CHK, CHK), 1)
    tstrict = (ci < ri).astype(jnp.float32)

    def chunk_ranks(ind):
        pref = jnp.zeros((1, E), jnp.float32)
        sls, rks = [], []
        for c in range(NCH):
            sl = lax.slice_in_dim(ind, c * CHK, (c + 1) * CHK, axis=0)
            loc = lax.dot_general(tstrict, sl, (((1,), (0,)), ((), ())),
                                  preferred_element_type=jnp.float32)
            rks.append(jnp.sum(sl * (loc + pref), axis=1, keepdims=True))
            sls.append(sl)
            pref = pref + jnp.sum(sl, axis=0, keepdims=True)
        return sls, rks, pref

    sl0, rk0, cnt0 = chunk_ranks(ind0)
    sl1, rk1, cnt1 = chunk_ranks(ind1)
    counts = cnt0 + cnt1
    seg = jnp.floor((counts + (BLK - 1)) / BLK) * BLK
    ea = lax.broadcasted_iota(jnp.int32, (E, E), 0)
    eb = lax.broadcasted_iota(jnp.int32, (E, E), 1)
    upper = (ea < eb).astype(jnp.float32)
    offs = lax.dot_general(seg, upper, (((1,), (0,)), ((), ())))
    base1 = offs + cnt0
    for c in range(NCH):
        o0 = jnp.sum(sl0[c] * offs, axis=1, keepdims=True)
        pos_ref[pl.ds(c * CHK, CHK), :] = (o0 + rk0[c]).astype(jnp.int32)
        o1 = jnp.sum(sl1[c] * base1, axis=1, keepdims=True)
        pos_ref[pl.ds(T + c * CHK, CHK), :] = (o1 + rk1[c]).astype(jnp.int32)
    ends = offs + seg
    bi = (lax.broadcasted_iota(jnp.int32, (NBLK, 1), 0) * BLK
          ).astype(jnp.float32)
    be = jnp.sum((bi >= ends).astype(jnp.float32), axis=1, keepdims=True)
    be_ref[...] = jnp.minimum(be, E - 1.0).astype(jnp.int32)


def _router(x, gate_w):
    return pl.pallas_call(
        _router_body,
        out_shape=[
            jax.ShapeDtypeStruct((A, 1), jnp.int32),
            jax.ShapeDtypeStruct((A, 1), jnp.float32),
            jax.ShapeDtypeStruct((NBLK, 1), jnp.int32),
            jax.ShapeDtypeStruct((T, D // 2), jnp.float32),
        ],
    )(x, gate_w)


DCH = 64
NCHD = (A // NW) // DCH


def _sc_dispatch_body(pos3_ref, tw3_ref, x_ref, xd_ref, ss_ref,
                      pvw2, tww2, pvw, tww, buf0, buf1, buf2,
                      gs0, gs1, gs2, ws0, ws1, ws2, ssem):
    wid = _worker_id()
    row0 = wid * 2
    pltpu.sync_copy(pos3_ref.at[pl.ds(row0, 2)], pvw2)
    pltpu.sync_copy(tw3_ref.at[pl.ds(row0, 2)], tww2)
    for j2 in range(2):
        for i in range(8):
            r4, c4 = j2 * 2 + i // 4, (i % 4) * 16
            pvw[r4, pl.ds(c4, 16)] = pvw2[j2, pl.ds(i * 16, 16)]
            tww[r4, pl.ds(c4, 16)] = tww2[j2, pl.ds(i * 16, 16)]
    sdescs = []
    for j in range(NCHD):
        sdescs.append(
            pltpu.async_copy(tww.at[j], ss_ref.at[pvw.at[j]], ssem))
    t0 = (wid * (A // NW)) & (T - 1)

    bufs, gsems, wsems = (buf0, buf1, buf2), (gs0, gs1, gs2), (ws0, ws1, ws2)
    gd = [None, None, None]
    wd = [None, None, None]

    def start(c):
        b = c % 3
        gd[b] = pltpu.async_copy(
            x_ref.at[pl.ds(pl.multiple_of(t0 + c * DCH, 8), DCH)],
            bufs[b], gsems[b])

    start(0)
    start(1)
    for c in range(NCHD):
        b = c % 3
        gd[b].wait()
        if c + 2 < NCHD:
            nb = (c + 2) % 3
            if wd[nb] is not None:
                wd[nb].wait()
            start(c + 2)
        wd[b] = pltpu.async_copy(bufs[b], xd_ref.at[pvw.at[c]], wsems[b])
    for b in range(3):
        if wd[b] is not None:
            wd[b].wait()
    for sd in sdescs:
        sd.wait()


@functools.lru_cache(maxsize=None)
def _sc_dispatch():
    return pl.kernel(
        _sc_dispatch_body,
        out_type=[
            jax.ShapeDtypeStruct((PAD, D // 2), jnp.float32),
            jax.ShapeDtypeStruct((PAD,), jnp.float32),
        ],
        mesh=_sc_mesh(),
        scratch_types=[
            pltpu.VMEM((2, 128), jnp.int32),
            pltpu.VMEM((2, 128), jnp.float32),
            pltpu.VMEM((NCHD, DCH), jnp.int32),
            pltpu.VMEM((NCHD, DCH), jnp.float32),
            pltpu.VMEM((DCH, D // 2), jnp.float32),
            pltpu.VMEM((DCH, D // 2), jnp.float32),
            pltpu.VMEM((DCH, D // 2), jnp.float32),
            pltpu.SemaphoreType.DMA,
            pltpu.SemaphoreType.DMA,
            pltpu.SemaphoreType.DMA,
            pltpu.SemaphoreType.DMA,
            pltpu.SemaphoreType.DMA,
            pltpu.SemaphoreType.DMA,
            pltpu.SemaphoreType.DMA,
        ],
    )


def _ffn_body(be_ref, xd_ref, wg_ref, wu_ref, wd_ref, ss_ref, y_ref):
    p = lax.bitcast_convert_type(xd_ref[...], jnp.uint32)
    xlo = lax.bitcast_convert_type(p << 16, jnp.float32)
    xhi = lax.bitcast_convert_type(p & jnp.uint32(0xFFFF0000), jnp.float32)
    xb = lax.concatenate(
        [xlo.astype(jnp.bfloat16), xhi.astype(jnp.bfloat16)], 1)
    g = lax.dot_general(xb, wg_ref[0], (((1,), (1,)), ((), ())),
                        preferred_element_type=jnp.float32)
    u = lax.dot_general(xb, wu_ref[0], (((1,), (1,)), ((), ())),
                        preferred_element_type=jnp.float32)
    h = (g * jax.nn.sigmoid(g) * u).astype(jnp.bfloat16)
    y = lax.dot_general(h, wd_ref[0], (((1,), (1,)), ((), ())),
                        preferred_element_type=jnp.float32)
    y_ref[...] = y * ss_ref[...].reshape(BLK, 1)


def _ffn(be, xd, w_gate, w_up, w_down, ss):
    grid_spec = pltpu.PrefetchScalarGridSpec(
        num_scalar_prefetch=1,
        grid=(NBLK,),
        in_specs=[
            pl.BlockSpec((BLK, D // 2), lambda i, be: (i, 0)),
            pl.BlockSpec((1, F, D), lambda i, be: (be[i], 0, 0)),
            pl.BlockSpec((1, F, D), lambda i, be: (be[i], 0, 0)),
            pl.BlockSpec((1, D, F), lambda i, be: (be[i], 0, 0)),
            pl.BlockSpec((BLK,), lambda i, be: (i,)),
        ],
        out_specs=pl.BlockSpec((BLK, D), lambda i, be: (i, 0)),
    )
    return pl.pallas_call(
        _ffn_body,
        grid_spec=grid_spec,
        out_shape=jax.ShapeDtypeStruct((PAD, D), jnp.float32),
        compiler_params=pltpu.CompilerParams(
            dimension_semantics=("arbitrary",),
        ),
    )(be, xd, w_gate, w_up, w_down, ss)


def _sc_combine_body(pos3_ref, y_ref, out_ref, p0w, p1w,
                     ya0, yb0, ya1, yb1, ga0, gb0, ga1, gb1, ws0, ws1):
    wid = _worker_id()
    tok0 = pl.multiple_of(wid * TPW, 8)
    pltpu.sync_copy(pos3_ref.at[wid], p0w)
    pltpu.sync_copy(pos3_ref.at[NW + wid], p1w)
    yas, ybs = (ya0, ya1), (yb0, yb1)
    gas, gbs, wss = (ga0, ga1), (gb0, gb1), (ws0, ws1)
    gda = [None, None]
    gdb = [None, None]
    wd = [None, None]
    NCC = TPW // CCH

    def start(c):
        b = c & 1
        ia = p0w.at[pl.ds(c * CCH, CCH)]
        ib = p1w.at[pl.ds(c * CCH, CCH)]
        gda[b] = pltpu.async_copy(y_ref.at[ia], yas[b], gas[b])
        gdb[b] = pltpu.async_copy(y_ref.at[ib], ybs[b], gbs[b])

    start(0)
    for c in range(NCC):
        b = c & 1
        gda[b].wait()
        gdb[b].wait()
        if c + 1 < NCC:
            if wd[1 - b] is not None:
                wd[1 - b].wait()
            start(c + 1)
        ya, yb = yas[b], ybs[b]

        def add(j, carry):
            off = pl.multiple_of(j * 16, 16)
            for r in range(CCH):
                ya[r, pl.ds(off, 16)] = (ya[r, pl.ds(off, 16)]
                                         + yb[r, pl.ds(off, 16)])
            return carry

        lax.fori_loop(0, D // 16, add, 0)
        wd[b] = pltpu.async_copy(
            ya, out_ref.at[pl.ds(tok0 + c * CCH, CCH)], wss[b])
    wd[0].wait()
    wd[1].wait()


@functools.lru_cache(maxsize=None)
def _sc_combine():
    return pl.kernel(
        _sc_combine_body,
        out_type=jax.ShapeDtypeStruct((T, D), jnp.float32),
        mesh=_sc_mesh(),
        scratch_types=[
            pltpu.VMEM((TPW,), jnp.int32),
            pltpu.VMEM((TPW,), jnp.int32),
            pltpu.VMEM((CCH, D), jnp.float32),
            pltpu.VMEM((CCH, D), jnp.float32),
            pltpu.VMEM((CCH, D), jnp.float32),
            pltpu.VMEM((CCH, D), jnp.float32),
            pltpu.SemaphoreType.DMA,
            pltpu.SemaphoreType.DMA,
            pltpu.SemaphoreType.DMA,
            pltpu.SemaphoreType.DMA,
            pltpu.SemaphoreType.DMA,
            pltpu.SemaphoreType.DMA,
        ],
    )


def kernel(hidden_states, gate_w, w_gate, w_up, w_down):
    B, S, d = hidden_states.shape
    x = hidden_states.reshape(-1, d)
    pos_all, tw_all, be, x16 = _router(x, gate_w)
    posr = pos_all.reshape(A // 128, 128)
    xd, ss = _sc_dispatch()(posr, tw_all.reshape(A // 128, 128), x16)
    y = _ffn(be.reshape(NBLK), xd,
             w_gate.astype(jnp.bfloat16), w_up.astype(jnp.bfloat16),
             w_down.astype(jnp.bfloat16), ss)
    out = _sc_combine()(posr, y)
    return out.reshape(B, S, d)


# speedup vs baseline: 1.7746x; 1.0083x over previous
import functools

import jax
import jax.numpy as jnp
from jax import lax
from jax.experimental import pallas as pl
from jax.experimental.pallas import tpu as pltpu
from jax.experimental.pallas import tpu_sc as plsc

E = 8
K = 2
D = 1024
F = 512
T = 4096
A = T * K
BLK = 512
PAD = A + E * BLK
NBLK = PAD // BLK
CHK = 512
NCH = T // CHK

NC, NS = 2, 16
NW = NC * NS
RPW = PAD // NW
TPW = T // NW
GCH = 32
CCH = 16


@functools.lru_cache(maxsize=None)
def _sc_mesh():
    return plsc.VectorSubcoreMesh(
        core_axis_name="c", subcore_axis_name="s",
        num_cores=NC, num_subcores=NS)


def _worker_id():
    return lax.axis_index("s") * NC + lax.axis_index("c")


def _router_body(x_ref, gw_ref, pos_ref, tw_ref, be_ref, x16_ref):
    x = x_ref[...]
    gw = gw_ref[...]
    logits = lax.dot_general(x, gw, (((1,), (1,)), ((), ())),
                             preferred_element_type=jnp.float32)
    scores = jax.nn.sigmoid(logits)
    ie = lax.broadcasted_iota(jnp.int32, (T, E), 1)
    m1 = jnp.max(scores, axis=1, keepdims=True)
    e0 = jnp.min(jnp.where(scores >= m1, ie, E), axis=1, keepdims=True)
    oh0 = ie == e0
    s2 = jnp.where(oh0, -1.0, scores)
    m2 = jnp.max(s2, axis=1, keepdims=True)
    e1 = jnp.min(jnp.where(s2 >= m2, ie, E), axis=1, keepdims=True)
    oh1 = ie == e1
    den = m1 + m2 + 1e-20
    tw0 = m1 / den
    tw1 = m2 / den
    RPC = CHK // 128
    for c in range(NCH):
        tw_ref[pl.ds(c * RPC, RPC), :] = (
            lax.slice_in_dim(tw0, c * CHK, (c + 1) * CHK, axis=0)
            .reshape(RPC, 128))
        tw_ref[pl.ds(T // 128 + c * RPC, RPC), :] = (
            lax.slice_in_dim(tw1, c * CHK, (c + 1) * CHK, axis=0)
            .reshape(RPC, 128))
    u = lax.bitcast_convert_type(x, jnp.uint32)
    ulo = lax.slice_in_dim(u, 0, D // 2, axis=1)
    uhi = lax.slice_in_dim(u, D // 2, D, axis=1)

    def rnd16(v):
        return (v + jnp.uint32(0x7FFF) + ((v >> 16) & jnp.uint32(1))) >> 16

    packed = rnd16(ulo) | (rnd16(uhi) << 16)
    x16_ref[...] = lax.bitcast_convert_type(packed, jnp.float32)

    ind0 = oh0.astype(jnp.float32)
    ind1 = oh1.astype(jnp.float32)
    ri = lax.broadcasted_iota(jnp.int32, (CHK, CHK), 0)
    ci = lax.broadcasted_iota(jnp.int32, (CHK, CHK), 1)
    tstrict = (ci < ri).astype(jnp.float32)

    def chunk_ranks(ind):
        pref = jnp.zeros((1, E), jnp.float32)
        sls, rks = [], []
        for c in range(NCH):
            sl = lax.slice_in_dim(ind, c * CHK, (c + 1) * CHK, axis=0)
            loc = lax.dot_general(tstrict, sl, (((1,), (0,)), ((), ())),
                                  preferred_element_type=jnp.float32)
            rks.append(jnp.sum(sl * (loc + pref), axis=1, keepdims=True))
            sls.append(sl)
            pref = pref + jnp.sum(sl, axis=0, keepdims=True)
        return sls, rks, pref

    sl0, rk0, cnt0 = chunk_ranks(ind0)
    sl1, rk1, cnt1 = chunk_ranks(ind1)
    counts = cnt0 + cnt1
    seg = jnp.floor((counts + (BLK - 1)) / BLK) * BLK
    ea = lax.broadcasted_iota(jnp.int32, (E, E), 0)
    eb = lax.broadcasted_iota(jnp.int32, (E, E), 1)
    upper = (ea < eb).astype(jnp.float32)
    offs = lax.dot_general(seg, upper, (((1,), (0,)), ((), ())))
    base1 = offs + cnt0
    for c in range(NCH):
        o0 = jnp.sum(sl0[c] * offs, axis=1, keepdims=True)
        pos_ref[pl.ds(c * RPC, RPC), :] = (
            (o0 + rk0[c]).astype(jnp.int32).reshape(RPC, 128))
        o1 = jnp.sum(sl1[c] * base1, axis=1, keepdims=True)
        pos_ref[pl.ds(T // 128 + c * RPC, RPC), :] = (
            (o1 + rk1[c]).astype(jnp.int32).reshape(RPC, 128))
    ends = offs + seg
    bi = (lax.broadcasted_iota(jnp.int32, (NBLK, 1), 0) * BLK
          ).astype(jnp.float32)
    be = jnp.sum((bi >= ends).astype(jnp.float32), axis=1, keepdims=True)
    be_ref[...] = jnp.minimum(be, E - 1.0).astype(jnp.int32)


def _router(x, gate_w):
    return pl.pallas_call(
        _router_body,
        out_shape=[
            jax.ShapeDtypeStruct((A // 128, 128), jnp.int32),
            jax.ShapeDtypeStruct((A // 128, 128), jnp.float32),
            jax.ShapeDtypeStruct((NBLK, 1), jnp.int32),
            jax.ShapeDtypeStruct((T, D // 2), jnp.float32),
        ],
    )(x, gate_w)


DCH = 64
NCHD = (A // NW) // DCH


def _sc_dispatch_body(pos3_ref, tw3_ref, x_ref, xd_ref, ss_ref,
                      pvw2, tww2, pvw, tww, buf0, buf1, buf2,
                      gs0, gs1, gs2, ws0, ws1, ws2, ssem):
    wid = _worker_id()
    row0 = wid * 2
    pltpu.sync_copy(pos3_ref.at[pl.ds(row0, 2)], pvw2)
    pltpu.sync_copy(tw3_ref.at[pl.ds(row0, 2)], tww2)
    for j2 in range(2):
        for i in range(8):
            r4, c4 = j2 * 2 + i // 4, (i % 4) * 16
            pvw[r4, pl.ds(c4, 16)] = pvw2[j2, pl.ds(i * 16, 16)]
            tww[r4, pl.ds(c4, 16)] = tww2[j2, pl.ds(i * 16, 16)]
    sdescs = []
    for j in range(NCHD):
        sdescs.append(
            pltpu.async_copy(tww.at[j], ss_ref.at[pvw.at[j]], ssem))
    t0 = (wid * (A // NW)) & (T - 1)

    bufs, gsems, wsems = (buf0, buf1, buf2), (gs0, gs1, gs2), (ws0, ws1, ws2)
    gd = [None, None, None]
    wd = [None, None, None]

    def start(c):
        b = c % 3
        gd[b] = pltpu.async_copy(
            x_ref.at[pl.ds(pl.multiple_of(t0 + c * DCH, 8), DCH)],
            bufs[b], gsems[b])

    start(0)
    start(1)
    for c in range(NCHD):
        b = c % 3
        gd[b].wait()
        if c + 2 < NCHD:
            nb = (c + 2) % 3
            if wd[nb] is not None:
                wd[nb].wait()
            start(c + 2)
        wd[b] = pltpu.async_copy(bufs[b], xd_ref.at[pvw.at[c]], wsems[b])
    for b in range(3):
        if wd[b] is not None:
            wd[b].wait()
    for sd in sdescs:
        sd.wait()


@functools.lru_cache(maxsize=None)
def _sc_dispatch():
    return pl.kernel(
        _sc_dispatch_body,
        out_type=[
            jax.ShapeDtypeStruct((PAD, D // 2), jnp.float32),
            jax.ShapeDtypeStruct((PAD,), jnp.float32),
        ],
        mesh=_sc_mesh(),
        scratch_types=[
            pltpu.VMEM((2, 128), jnp.int32),
            pltpu.VMEM((2, 128), jnp.float32),
            pltpu.VMEM((NCHD, DCH), jnp.int32),
            pltpu.VMEM((NCHD, DCH), jnp.float32),
            pltpu.VMEM((DCH, D // 2), jnp.float32),
            pltpu.VMEM((DCH, D // 2), jnp.float32),
            pltpu.VMEM((DCH, D // 2), jnp.float32),
            pltpu.SemaphoreType.DMA,
            pltpu.SemaphoreType.DMA,
            pltpu.SemaphoreType.DMA,
            pltpu.SemaphoreType.DMA,
            pltpu.SemaphoreType.DMA,
            pltpu.SemaphoreType.DMA,
            pltpu.SemaphoreType.DMA,
        ],
    )


def _ffn_body(be_ref, xd_ref, wg_ref, wu_ref, wd_ref, ss_ref, y_ref):
    p = lax.bitcast_convert_type(xd_ref[...], jnp.uint32)
    xlo = lax.bitcast_convert_type(p << 16, jnp.float32)
    xhi = lax.bitcast_convert_type(p & jnp.uint32(0xFFFF0000), jnp.float32)
    xb = lax.concatenate(
        [xlo.astype(jnp.bfloat16), xhi.astype(jnp.bfloat16)], 1)
    g = lax.dot_general(xb, wg_ref[0], (((1,), (1,)), ((), ())),
                        preferred_element_type=jnp.float32)
    u = lax.dot_general(xb, wu_ref[0], (((1,), (1,)), ((), ())),
                        preferred_element_type=jnp.float32)
    h = (g * jax.nn.sigmoid(g) * u).astype(jnp.bfloat16)
    y = lax.dot_general(h, wd_ref[0], (((1,), (1,)), ((), ())),
                        preferred_element_type=jnp.float32)
    y_ref[...] = y * ss_ref[...].reshape(BLK, 1)


def _ffn(be, xd, w_gate, w_up, w_down, ss):
    grid_spec = pltpu.PrefetchScalarGridSpec(
        num_scalar_prefetch=1,
        grid=(NBLK,),
        in_specs=[
            pl.BlockSpec((BLK, D // 2), lambda i, be: (i, 0)),
            pl.BlockSpec((1, F, D), lambda i, be: (be[i], 0, 0)),
            pl.BlockSpec((1, F, D), lambda i, be: (be[i], 0, 0)),
            pl.BlockSpec((1, D, F), lambda i, be: (be[i], 0, 0)),
            pl.BlockSpec((BLK,), lambda i, be: (i,)),
        ],
        out_specs=pl.BlockSpec((BLK, D), lambda i, be: (i, 0)),
    )
    return pl.pallas_call(
        _ffn_body,
        grid_spec=grid_spec,
        out_shape=jax.ShapeDtypeStruct((PAD, D), jnp.float32),
        compiler_params=pltpu.CompilerParams(
            dimension_semantics=("arbitrary",),
        ),
    )(be, xd, w_gate, w_up, w_down, ss)


def _sc_combine_body(pos3_ref, y_ref, out_ref, p0w, p1w,
                     ya0, yb0, ya1, yb1, ga0, gb0, ga1, gb1, ws0, ws1):
    wid = _worker_id()
    tok0 = pl.multiple_of(wid * TPW, 8)
    pltpu.sync_copy(pos3_ref.at[wid], p0w)
    pltpu.sync_copy(pos3_ref.at[NW + wid], p1w)
    yas, ybs = (ya0, ya1), (yb0, yb1)
    gas, gbs, wss = (ga0, ga1), (gb0, gb1), (ws0, ws1)
    gda = [None, None]
    gdb = [None, None]
    wd = [None, None]
    NCC = TPW // CCH

    def start(c):
        b = c & 1
        ia = p0w.at[pl.ds(c * CCH, CCH)]
        ib = p1w.at[pl.ds(c * CCH, CCH)]
        gda[b] = pltpu.async_copy(y_ref.at[ia], yas[b], gas[b])
        gdb[b] = pltpu.async_copy(y_ref.at[ib], ybs[b], gbs[b])

    start(0)
    for c in range(NCC):
        b = c & 1
        gda[b].wait()
        gdb[b].wait()
        if c + 1 < NCC:
            if wd[1 - b] is not None:
                wd[1 - b].wait()
            start(c + 1)
        ya, yb = yas[b], ybs[b]

        def add(j, carry):
            off = pl.multiple_of(j * 16, 16)
            for r in range(CCH):
                ya[r, pl.ds(off, 16)] = (ya[r, pl.ds(off, 16)]
                                         + yb[r, pl.ds(off, 16)])
            return carry

        lax.fori_loop(0, D // 16, add, 0)
        wd[b] = pltpu.async_copy(
            ya, out_ref.at[pl.ds(tok0 + c * CCH, CCH)], wss[b])
    wd[0].wait()
    wd[1].wait()


@functools.lru_cache(maxsize=None)
def _sc_combine():
    return pl.kernel(
        _sc_combine_body,
        out_type=jax.ShapeDtypeStruct((T, D), jnp.float32),
        mesh=_sc_mesh(),
        scratch_types=[
            pltpu.VMEM((TPW,), jnp.int32),
            pltpu.VMEM((TPW,), jnp.int32),
            pltpu.VMEM((CCH, D), jnp.float32),
            pltpu.VMEM((CCH, D), jnp.float32),
            pltpu.VMEM((CCH, D), jnp.float32),
            pltpu.VMEM((CCH, D), jnp.float32),
            pltpu.SemaphoreType.DMA,
            pltpu.SemaphoreType.DMA,
            pltpu.SemaphoreType.DMA,
            pltpu.SemaphoreType.DMA,
            pltpu.SemaphoreType.DMA,
            pltpu.SemaphoreType.DMA,
        ],
    )


def kernel(hidden_states, gate_w, w_gate, w_up, w_down):
    B, S, d = hidden_states.shape
    x = hidden_states.reshape(-1, d)
    posr, twr, be, x16 = _router(x, gate_w)
    xd, ss = _sc_dispatch()(posr, twr, x16)
    y = _ffn(be.reshape(NBLK), xd,
             w_gate.astype(jnp.bfloat16), w_up.astype(jnp.bfloat16),
             w_down.astype(jnp.bfloat16), ss)
    out = _sc_combine()(posr, y)
    return out.reshape(B, S, d)
